# baseline (speedup 1.0000x reference)
import functools

import jax
import jax.numpy as jnp
from jax import lax
from jax.experimental import pallas as pl
from jax.experimental.pallas import tpu as pltpu, tpu_sc as plsc

_AL = 2048
_NC = 81


def _dense_body(ablocks, cd_ref, ct_ref, s_ref, npos_ref, acc_ref):
    b = pl.program_id(0)
    a = pl.program_id(1)

    @pl.when(jnp.logical_and(b == 0, a == 0))
    def _():
        acc_ref[...] = jnp.zeros_like(acc_ref)

    @pl.when(a == 0)
    def _():
        npos_ref[...] = jnp.zeros_like(npos_ref)

    x = jnp.transpose(cd_ref[0])
    ct = ct_ref[0, 0]
    m = jnp.max(x, axis=0, keepdims=True)
    e = jnp.exp(x - m)
    lse = jnp.log(jnp.sum(e, axis=0, keepdims=True)) + m
    sub_iota = jax.lax.broadcasted_iota(jnp.int32, x.shape, 0)
    tgt = jnp.sum(jnp.where(sub_iota == ct, x, 0.0), axis=0, keepdims=True)
    ce = lse - tgt

    pos = ct > 0
    posf = pos.astype(jnp.float32)
    s_ref[0, 0] = jnp.where(pos, 0.0, ce)

    pos_ce_part = jnp.sum(ce * posf)
    npos_part = jnp.sum(posf)

    npos_ref[0] += npos_part
    lane = jax.lax.broadcasted_iota(jnp.int32, (1, 128), 1)
    acc_ref[...] += jnp.where(lane == 1, pos_ce_part, 0.0)


def _loc_body(lt_ref, ld_ref, ct_ref, acc_ref):
    b = pl.program_id(0)
    a = pl.program_id(1)

    @pl.when(jnp.logical_and(b == 0, a == 0))
    def _():
        acc_ref[...] = jnp.zeros_like(acc_ref)

    posf = (ct_ref[0] > 0).astype(jnp.float32)
    diff = ld_ref[0] - lt_ref[0]
    ad = jnp.abs(diff)
    sl1 = jnp.where(ad < 1.0, 0.5 * diff * diff, ad - 0.5)
    loss_l_part = jnp.sum(jnp.sum(sl1, axis=0, keepdims=True) * posf)
    lane = jax.lax.broadcasted_iota(jnp.int32, (1, 128), 1)
    acc_ref[...] += jnp.where(lane == 0, loss_l_part, 0.0)


def _sc_mine_body(p_pad, num_priors,
                  s_hbm, npos_hbm, out_hbm,
                  sv, nv, h1c, h1v, h2c, h2v, h3c, h3v, rv, tmpi, tmpf):
    row = lax.axis_index("s") * 2 + lax.axis_index("c")
    nchunk = p_pad // 16

    pltpu.sync_copy(s_hbm.at[pl.ds(row * p_pad, p_pad)], sv)
    pltpu.sync_copy(npos_hbm, nv)

    iota = lax.iota(jnp.int32, 16)
    zeros16 = jnp.zeros((16,), jnp.int32)

    def allsum_i(v):
        for d in (1, 2, 4, 8):
            tmpi[...] = v
            v = v + plsc.load_gather(tmpi, [jnp.bitwise_xor(iota, d)])
        return v

    def suffix_i(v):
        for d in (1, 2, 4, 8):
            tmpi[...] = v
            g = plsc.load_gather(tmpi, [jnp.minimum(iota + d, 15)])
            v = v + jnp.where(iota + d < 16, g, 0)
        return v

    def suffix_f(v):
        for d in (1, 2, 4, 8):
            tmpf[...] = v
            g = plsc.load_gather(tmpf, [jnp.minimum(iota + d, 15)])
            v = v + jnp.where(iota + d < 16, g, 0.0)
        return v

    def pick_i(v, idx):
        tmpi[...] = v
        return plsc.load_gather(tmpi, [idx])

    def pick_f(v, idx):
        tmpf[...] = v
        return plsc.load_gather(tmpf, [idx])

    npos_w = plsc.load_gather(nv, [zeros16 + row])
    k = jnp.minimum(3 * npos_w, num_priors - 1)

    zi = jnp.zeros((16,), jnp.int32)
    zf = jnp.zeros((16,), jnp.float32)
    ones = jnp.ones((16,), jnp.int32)

    def zero12(j, carry):
        h1c[pl.ds(j * 16, 16)] = zi
        h1v[pl.ds(j * 16, 16)] = zf
        h2c[pl.ds(j * 16, 16)] = zi
        h2v[pl.ds(j * 16, 16)] = zf
        return carry

    lax.fori_loop(0, 128, zero12, 0)

    def zero3(j, carry):
        h3c[pl.ds(j * 16, 16)] = zi
        h3v[pl.ds(j * 16, 16)] = zf
        return carry

    lax.fori_loop(0, 32, zero3, 0)

    def pass1(j, carry):
        x = sv[pl.ds(j * 16, 16)]
        b = plsc.bitcast(x, jnp.int32)
        i1 = lax.shift_right_logical(b, 20)
        plsc.addupdate_scatter(h1c, [i1], ones)
        plsc.addupdate_scatter(h1v, [i1], x)
        return carry

    lax.fori_loop(0, nchunk, pass1, 0)

    def scan_level(hc, hv, nchunks, cnt0, val0):
        def body(t, carry):
            done, bsel, cnt, val, base = carry
            j = nchunks - 1 - t
            c = hc[pl.ds(j * 16, 16)]
            v = hv[pl.ds(j * 16, 16)]
            suf_c = suffix_i(c)
            suf_v = suffix_f(v)
            cvec = cnt + suf_c
            mask = cvec >= k
            nm = allsum_i(mask.astype(jnp.int32))
            found = jnp.logical_and(done == 0, nm > 0)
            istar = nm - 1
            isafe = jnp.minimum(jnp.maximum(istar, 0) + 1, 15)
            tot = pick_i(suf_c, zeros16)
            vtot = pick_f(suf_v, zeros16)
            above_c = jnp.where(istar >= 15, 0, pick_i(suf_c, isafe))
            above_v = jnp.where(istar >= 15, 0.0, pick_f(suf_v, isafe))
            nb = jnp.where(found, base + istar, bsel)
            ncnt = jnp.where(found, cnt + above_c,
                             jnp.where(done == 0, cnt + tot, cnt))
            nval = jnp.where(found, val + above_v,
                             jnp.where(done == 0, val + vtot, val))
            ndone = jnp.where(found, 1, done)
            return (ndone, nb, ncnt, nval, base - 16)

        base0 = jnp.full((16,), (nchunks - 1) * 16, jnp.int32)
        out = lax.fori_loop(0, nchunks, body,
                            (zeros16, zeros16, cnt0, val0, base0))
        return out[0], out[1], out[2], out[3]

    _, b1, cnt1, val1 = scan_level(h1c, h1v, 128, zeros16,
                                   jnp.zeros((16,), jnp.float32))

    def pass2(j, carry):
        x = sv[pl.ds(j * 16, 16)]
        b = plsc.bitcast(x, jnp.int32)
        msk = lax.shift_right_logical(b, 20) == b1
        i2 = jnp.bitwise_and(lax.shift_right_logical(b, 9), 0x7FF)
        plsc.addupdate_scatter(h2c, [i2], ones, mask=msk)
        plsc.addupdate_scatter(h2v, [i2], x, mask=msk)
        return carry

    lax.fori_loop(0, nchunk, pass2, 0)
    _, b2, cnt2, val2 = scan_level(h2c, h2v, 128, cnt1, val1)

    pref = jnp.bitwise_or(lax.shift_left(b1, 11), b2)

    def pass3(j, carry):
        x = sv[pl.ds(j * 16, 16)]
        b = plsc.bitcast(x, jnp.int32)
        msk = lax.shift_right_logical(b, 9) == pref
        i3 = jnp.bitwise_and(b, 0x1FF)
        plsc.addupdate_scatter(h3c, [i3], ones, mask=msk)
        plsc.addupdate_scatter(h3v, [i3], x, mask=msk)
        return carry

    lax.fori_loop(0, nchunk, pass3, 0)
    _, b3, c_gt, sum_gt = scan_level(h3c, h3v, 32, cnt2, val2)

    v_bits = jnp.bitwise_or(jnp.bitwise_or(
        lax.shift_left(b1, 20), lax.shift_left(b2, 9)), b3)
    v_f = plsc.bitcast(v_bits, jnp.float32)
    extra = jnp.where(k > c_gt, (k - c_gt).astype(jnp.float32) * v_f, 0.0)
    neg = sum_gt + extra

    rv[...] = neg
    pltpu.sync_copy(rv.at[pl.ds(0, 8)], out_hbm.at[pl.ds(row * 8, 8)])


def _fin_body(npos_ref, acc_ref, accl_ref, negc_ref, out_l_ref, out_c_ref):
    num_pos = npos_ref[:, :1]
    n_total = jnp.sum(num_pos)
    acc = acc_ref[...]
    lane = jax.lax.broadcasted_iota(jnp.int32, acc.shape, 1)
    loss_l_total = jnp.sum(jnp.where(lane == 0, accl_ref[...], 0.0))
    pos_ce_total = jnp.sum(jnp.where(lane == 1, acc, 0.0))
    negc = negc_ref[...]
    l256 = jax.lax.broadcasted_iota(jnp.int32, negc.shape, 1)
    neg_total = jnp.sum(jnp.where(l256 % 8 == 0, negc, 0.0))
    loss_l = loss_l_total / n_total
    loss_c = (pos_ce_total + neg_total) / n_total
    out_l_ref[...] = jnp.full(out_l_ref.shape, loss_l)
    out_c_ref[...] = jnp.full(out_c_ref.shape, loss_c)


def kernel(loc_t, loc_data, conf_t, conf_data):
    num, num_priors, nc = conf_data.shape
    al = 2000
    ablocks = num_priors // al
    alp = 2048
    pblocks = -(-num_priors // alp)
    p_pad = pblocks * alp
    pad = p_pad - num_priors

    ct4 = conf_t.reshape(num, ablocks, 1, al)

    s_out, npos_out, acc_out = pl.pallas_call(
        functools.partial(_dense_body, ablocks),
        grid=(num, ablocks),
        in_specs=[
            pl.BlockSpec((1, al, nc), lambda b, a: (b, a, 0)),
            pl.BlockSpec((1, 1, 1, al), lambda b, a: (b, a, 0, 0)),
        ],
        out_specs=[
            pl.BlockSpec((1, 1, 1, al), lambda b, a: (b, a, 0, 0)),
            pl.BlockSpec((1, 1, 128), lambda b, a: (b, 0, 0)),
            pl.BlockSpec((1, 128), lambda b, a: (0, 0)),
        ],
        out_shape=[
            jax.ShapeDtypeStruct((num, ablocks, 1, al), jnp.float32),
            jax.ShapeDtypeStruct((num, 1, 128), jnp.float32),
            jax.ShapeDtypeStruct((1, 128), jnp.float32),
        ],
    )(conf_data, ct4)

    lt = jnp.pad(jnp.transpose(loc_t, (0, 2, 1)), ((0, 0), (0, 0), (0, pad)))
    ld = jnp.pad(jnp.transpose(loc_data, (0, 2, 1)), ((0, 0), (0, 0), (0, pad)))
    ctp = jnp.pad(conf_t, ((0, 0), (0, pad))).reshape(num, 1, p_pad)

    acc_l = pl.pallas_call(
        _loc_body,
        grid=(num, pblocks),
        in_specs=[
            pl.BlockSpec((1, 4, alp), lambda b, a: (b, 0, a)),
            pl.BlockSpec((1, 4, alp), lambda b, a: (b, 0, a)),
            pl.BlockSpec((1, 1, alp), lambda b, a: (b, 0, a)),
        ],
        out_specs=pl.BlockSpec((1, 128), lambda b, a: (0, 0)),
        out_shape=jax.ShapeDtypeStruct((1, 128), jnp.float32),
    )(lt, ld, ctp)

    s_flat = s_out.reshape(num * num_priors)
    npos = npos_out.reshape(num, 128)
    npos_i32 = npos[:, 0].astype(jnp.int32)

    mesh = plsc.VectorSubcoreMesh(core_axis_name="c", subcore_axis_name="s")
    negc = pl.kernel(
        functools.partial(_sc_mine_body, num_priors, num_priors),
        mesh=mesh,
        compiler_params=pltpu.CompilerParams(needs_layout_passes=False),
        out_type=jax.ShapeDtypeStruct((num * 8,), jnp.float32),
        scratch_types=[
            pltpu.VMEM((num_priors,), jnp.float32),
            pltpu.VMEM((num,), jnp.int32),
            pltpu.VMEM((2048,), jnp.int32),
            pltpu.VMEM((2048,), jnp.float32),
            pltpu.VMEM((2048,), jnp.int32),
            pltpu.VMEM((2048,), jnp.float32),
            pltpu.VMEM((512,), jnp.int32),
            pltpu.VMEM((512,), jnp.float32),
            pltpu.VMEM((16,), jnp.float32),
            pltpu.VMEM((16,), jnp.int32),
            pltpu.VMEM((16,), jnp.float32),
        ],
    )(s_flat, npos_i32)

    out_l, out_c = pl.pallas_call(
        _fin_body,
        out_shape=[
            jax.ShapeDtypeStruct((1, 128), jnp.float32),
            jax.ShapeDtypeStruct((1, 128), jnp.float32),
        ],
    )(npos, acc_out, acc_l, negc.reshape(1, num * 8))

    return (out_l[0, 0], out_c[0, 0])


# speedup vs baseline: 1.1588x; 1.1588x over previous
import functools

import jax
import jax.numpy as jnp
from jax import lax
from jax.experimental import pallas as pl
from jax.experimental.pallas import tpu as pltpu, tpu_sc as plsc

_AL = 2048
_NC = 81


def _dense_body(ablocks, num_priors, cd_ref, ct_ref, lt_ref, ld_ref,
                s_ref, npos_ref, acc_ref):
    b = pl.program_id(0)
    a = pl.program_id(1)

    @pl.when(jnp.logical_and(b == 0, a == 0))
    def _():
        acc_ref[...] = jnp.zeros_like(acc_ref)

    @pl.when(a == 0)
    def _():
        npos_ref[...] = jnp.zeros_like(npos_ref)

    x = cd_ref[0]
    ct = ct_ref[0]
    m = jnp.max(x, axis=0, keepdims=True)
    e = jnp.exp(x - m)
    lse = jnp.log(jnp.sum(e, axis=0, keepdims=True)) + m
    sub_iota = jax.lax.broadcasted_iota(jnp.int32, x.shape, 0)
    tgt = jnp.sum(jnp.where(sub_iota == ct, x, 0.0), axis=0, keepdims=True)
    ce = lse - tgt

    pos = ct > 0
    posf = pos.astype(jnp.float32)
    ai = jax.lax.broadcasted_iota(jnp.int32, ct.shape, 1) + a * _AL
    dead = jnp.logical_or(pos, ai >= num_priors)
    s_ref[0] = jnp.where(dead, 0.0, ce)

    diff = ld_ref[0] - lt_ref[0]
    ad = jnp.abs(diff)
    sl1 = jnp.where(ad < 1.0, 0.5 * diff * diff, ad - 0.5)
    loss_l_part = jnp.sum(jnp.sum(sl1, axis=0, keepdims=True) * posf)
    pos_ce_part = jnp.sum(ce * posf)
    npos_part = jnp.sum(posf)

    npos_ref[0] += npos_part
    lane = jax.lax.broadcasted_iota(jnp.int32, (1, 128), 1)
    acc_ref[...] += (jnp.where(lane == 0, loss_l_part, 0.0)
                     + jnp.where(lane == 1, pos_ce_part, 0.0))


def _sc_mine_body(p_pad, num_priors,
                  s_hbm, npos_hbm, out_hbm,
                  sv, nv, h1c, h2c, h3c, rv, tmpi):
    row = lax.axis_index("s") * 2 + lax.axis_index("c")
    nchunk4 = p_pad // 64

    pltpu.sync_copy(s_hbm.at[pl.ds(row * p_pad, p_pad)], sv)
    pltpu.sync_copy(npos_hbm, nv)

    iota = lax.iota(jnp.int32, 16)
    zeros16 = jnp.zeros((16,), jnp.int32)

    def allsum_i(v):
        for d in (1, 2, 4, 8):
            tmpi[...] = v
            v = v + plsc.load_gather(tmpi, [jnp.bitwise_xor(iota, d)])
        return v

    def suffix_i(v):
        for d in (1, 2, 4, 8):
            tmpi[...] = v
            g = plsc.load_gather(tmpi, [jnp.minimum(iota + d, 15)])
            v = v + jnp.where(iota + d < 16, g, 0)
        return v

    def pick_i(v, idx):
        tmpi[...] = v
        return plsc.load_gather(tmpi, [idx])

    npos_w = plsc.load_gather(nv, [zeros16 + row])
    k = jnp.minimum(3 * npos_w, num_priors - 1)

    zi = jnp.zeros((16,), jnp.int32)
    ones = jnp.ones((16,), jnp.int32)

    def zero12(j, carry):
        h1c[pl.ds(j * 16, 16)] = zi
        h2c[pl.ds(j * 16, 16)] = zi
        return carry

    lax.fori_loop(0, 128, zero12, 0)

    def zero3(j, carry):
        h3c[pl.ds(j * 16, 16)] = zi
        return carry

    lax.fori_loop(0, 32, zero3, 0)

    def pass1(j, carry):
        for u in range(4):
            x = sv[pl.ds(j * 64 + u * 16, 16)]
            b = plsc.bitcast(x, jnp.int32)
            plsc.addupdate_scatter(h1c, [lax.shift_right_logical(b, 20)],
                                   ones)
        return carry

    lax.fori_loop(0, nchunk4, pass1, 0)

    def scan_level(hc, nchunks, cnt0):
        def body(t, carry):
            done, bsel, cnt, base = carry
            j = nchunks - 1 - t
            c = hc[pl.ds(j * 16, 16)]
            suf_c = suffix_i(c)
            mask = (cnt + suf_c) >= k
            nm = allsum_i(mask.astype(jnp.int32))
            found = jnp.logical_and(done == 0, nm > 0)
            istar = nm - 1
            isafe = jnp.minimum(jnp.maximum(istar, 0) + 1, 15)
            tot = pick_i(suf_c, zeros16)
            above_c = jnp.where(istar >= 15, 0, pick_i(suf_c, isafe))
            nb = jnp.where(found, base + istar, bsel)
            ncnt = jnp.where(found, cnt + above_c,
                             jnp.where(done == 0, cnt + tot, cnt))
            ndone = jnp.where(found, 1, done)
            return (ndone, nb, ncnt, base - 16)

        base0 = jnp.full((16,), (nchunks - 1) * 16, jnp.int32)
        out = lax.fori_loop(0, nchunks, body,
                            (zeros16, zeros16, cnt0, base0))
        return out[1], out[2]

    b1, cnt1 = scan_level(h1c, 128, zeros16)

    def pass2(j, carry):
        for u in range(4):
            x = sv[pl.ds(j * 64 + u * 16, 16)]
            b = plsc.bitcast(x, jnp.int32)
            msk = lax.shift_right_logical(b, 20) == b1
            i2 = jnp.bitwise_and(lax.shift_right_logical(b, 9), 0x7FF)
            plsc.addupdate_scatter(h2c, [i2], ones, mask=msk)
        return carry

    lax.fori_loop(0, nchunk4, pass2, 0)
    b2, cnt2 = scan_level(h2c, 128, cnt1)

    pref = jnp.bitwise_or(lax.shift_left(b1, 11), b2)

    def pass3(j, carry):
        for u in range(4):
            x = sv[pl.ds(j * 64 + u * 16, 16)]
            b = plsc.bitcast(x, jnp.int32)
            msk = lax.shift_right_logical(b, 9) == pref
            i3 = jnp.bitwise_and(b, 0x1FF)
            plsc.addupdate_scatter(h3c, [i3], ones, mask=msk)
        return carry

    lax.fori_loop(0, nchunk4, pass3, 0)
    b3, _ = scan_level(h3c, 32, cnt2)

    v_bits = jnp.bitwise_or(jnp.bitwise_or(
        lax.shift_left(b1, 20), lax.shift_left(b2, 9)), b3)

    rv[...] = v_bits
    pltpu.sync_copy(rv.at[pl.ds(0, 8)], out_hbm.at[pl.ds(row * 8, 8)])


def _fin_body(num_priors, s_ref, bits_ref, vb_ref, npos_ref, acc_ref,
              out_l_ref, out_c_ref):
    s = s_ref[...]
    bits = bits_ref[...]
    v = vb_ref[:, :1]
    num_pos = npos_ref[:, :1]
    num_neg = jnp.minimum(3 * num_pos.astype(jnp.int32), num_priors - 1)

    c_gt = jnp.sum((bits > v).astype(jnp.int32), axis=1, keepdims=True)
    sum_gt = jnp.sum(jnp.where(bits > v, s, 0.0), axis=1, keepdims=True)
    v_f = jnp.max(jnp.where(bits == v, s, 0.0), axis=1, keepdims=True)
    extra = jnp.where(num_neg > c_gt,
                      (num_neg - c_gt).astype(jnp.float32) * v_f, 0.0)
    neg_contrib = jnp.sum(sum_gt + extra)

    n_total = jnp.sum(num_pos)
    acc = acc_ref[...]
    lane = jax.lax.broadcasted_iota(jnp.int32, acc.shape, 1)
    loss_l_total = jnp.sum(jnp.where(lane == 0, acc, 0.0))
    pos_ce_total = jnp.sum(jnp.where(lane == 1, acc, 0.0))
    loss_l = loss_l_total / n_total
    loss_c = (pos_ce_total + neg_contrib) / n_total
    out_l_ref[...] = jnp.full(out_l_ref.shape, loss_l)
    out_c_ref[...] = jnp.full(out_c_ref.shape, loss_c)


def kernel(loc_t, loc_data, conf_t, conf_data):
    num, num_priors, nc = conf_data.shape
    ablocks = -(-num_priors // _AL)
    p_pad = ablocks * _AL
    pad = p_pad - num_priors

    cd = jnp.pad(jnp.transpose(conf_data, (0, 2, 1)), ((0, 0), (0, 0), (0, pad)))
    lt = jnp.pad(jnp.transpose(loc_t, (0, 2, 1)), ((0, 0), (0, 0), (0, pad)))
    ld = jnp.pad(jnp.transpose(loc_data, (0, 2, 1)), ((0, 0), (0, 0), (0, pad)))
    ct = jnp.pad(conf_t, ((0, 0), (0, pad))).reshape(num, 1, p_pad)

    s_out, npos_out, acc_out = pl.pallas_call(
        functools.partial(_dense_body, ablocks, num_priors),
        grid=(num, ablocks),
        in_specs=[
            pl.BlockSpec((1, nc, _AL), lambda b, a: (b, 0, a)),
            pl.BlockSpec((1, 1, _AL), lambda b, a: (b, 0, a)),
            pl.BlockSpec((1, 4, _AL), lambda b, a: (b, 0, a)),
            pl.BlockSpec((1, 4, _AL), lambda b, a: (b, 0, a)),
        ],
        out_specs=[
            pl.BlockSpec((1, 1, _AL), lambda b, a: (b, 0, a)),
            pl.BlockSpec((1, 1, 128), lambda b, a: (b, 0, 0)),
            pl.BlockSpec((1, 128), lambda b, a: (0, 0)),
        ],
        out_shape=[
            jax.ShapeDtypeStruct((num, 1, p_pad), jnp.float32),
            jax.ShapeDtypeStruct((num, 1, 128), jnp.float32),
            jax.ShapeDtypeStruct((1, 128), jnp.float32),
        ],
    )(cd, ct, lt, ld)

    s_flat = s_out.reshape(num * p_pad)
    npos = npos_out.reshape(num, 128)
    npos_i32 = npos[:, 0].astype(jnp.int32)

    mesh = plsc.VectorSubcoreMesh(core_axis_name="c", subcore_axis_name="s")
    vbits = pl.kernel(
        functools.partial(_sc_mine_body, p_pad, num_priors),
        mesh=mesh,
        compiler_params=pltpu.CompilerParams(needs_layout_passes=False),
        out_type=jax.ShapeDtypeStruct((num * 8,), jnp.int32),
        scratch_types=[
            pltpu.VMEM((p_pad,), jnp.float32),
            pltpu.VMEM((num,), jnp.int32),
            pltpu.VMEM((2048,), jnp.int32),
            pltpu.VMEM((2048,), jnp.int32),
            pltpu.VMEM((512,), jnp.int32),
            pltpu.VMEM((16,), jnp.int32),
            pltpu.VMEM((16,), jnp.int32),
        ],
    )(s_flat, npos_i32)

    s = s_out.reshape(num, p_pad)
    bits = jax.lax.bitcast_convert_type(s, jnp.int32)

    out_l, out_c = pl.pallas_call(
        functools.partial(_fin_body, num_priors),
        out_shape=[
            jax.ShapeDtypeStruct((1, 128), jnp.float32),
            jax.ShapeDtypeStruct((1, 128), jnp.float32),
        ],
    )(s, bits, vbits.reshape(num, 8), npos, acc_out)

    return (out_l[0, 0], out_c[0, 0])


# speedup vs baseline: 1.2959x; 1.1184x over previous
import functools

import jax
import jax.numpy as jnp
from jax import lax
from jax.experimental import pallas as pl
from jax.experimental.pallas import tpu as pltpu, tpu_sc as plsc

_AL = 2048
_NC = 81


def _dense_body(ablocks, num_priors, cd_ref, ct_ref, lt_ref, ld_ref,
                s_ref, npos_ref, acc_ref):
    b = pl.program_id(0)
    a = pl.program_id(1)

    @pl.when(jnp.logical_and(b == 0, a == 0))
    def _():
        acc_ref[...] = jnp.zeros_like(acc_ref)

    @pl.when(a == 0)
    def _():
        npos_ref[...] = jnp.zeros_like(npos_ref)

    x = cd_ref[0].astype(jnp.float32)
    ct = ct_ref[0]
    m = jnp.max(x, axis=0, keepdims=True)
    e = jnp.exp(x - m)
    lse = jnp.log(jnp.sum(e, axis=0, keepdims=True)) + m
    sub_iota = jax.lax.broadcasted_iota(jnp.int32, x.shape, 0)
    tgt = jnp.sum(jnp.where(sub_iota == ct, x, 0.0), axis=0, keepdims=True)
    ce = lse - tgt

    pos = ct > 0
    posf = pos.astype(jnp.float32)
    ai = jax.lax.broadcasted_iota(jnp.int32, ct.shape, 1) + a * _AL
    dead = jnp.logical_or(pos, ai >= num_priors)
    s_ref[0] = jnp.where(dead, 0.0, ce)

    diff = ld_ref[0] - lt_ref[0]
    ad = jnp.abs(diff)
    sl1 = jnp.where(ad < 1.0, 0.5 * diff * diff, ad - 0.5)
    loss_l_part = jnp.sum(jnp.sum(sl1, axis=0, keepdims=True) * posf)
    pos_ce_part = jnp.sum(ce * posf)
    npos_part = jnp.sum(posf)

    npos_ref[0] += npos_part
    lane = jax.lax.broadcasted_iota(jnp.int32, (1, 128), 1)
    acc_ref[...] += (jnp.where(lane == 0, loss_l_part, 0.0)
                     + jnp.where(lane == 1, pos_ce_part, 0.0))


def _sc_mine_body(p_pad, num_priors,
                  s_hbm, npos_hbm, out_hbm,
                  sv, nv, h1c, h2c, h3c, rv, tmpi):
    row = lax.axis_index("s") * 2 + lax.axis_index("c")
    nchunk4 = p_pad // 64

    pltpu.sync_copy(s_hbm.at[pl.ds(row * p_pad, p_pad)], sv)
    pltpu.sync_copy(npos_hbm, nv)

    iota = lax.iota(jnp.int32, 16)
    zeros16 = jnp.zeros((16,), jnp.int32)

    def allsum_i(v):
        for d in (1, 2, 4, 8):
            tmpi[...] = v
            v = v + plsc.load_gather(tmpi, [jnp.bitwise_xor(iota, d)])
        return v

    def suffix_i(v):
        for d in (1, 2, 4, 8):
            tmpi[...] = v
            g = plsc.load_gather(tmpi, [jnp.minimum(iota + d, 15)])
            v = v + jnp.where(iota + d < 16, g, 0)
        return v

    def pick_i(v, idx):
        tmpi[...] = v
        return plsc.load_gather(tmpi, [idx])

    npos_w = plsc.load_gather(nv, [zeros16 + row])
    k = jnp.minimum(3 * npos_w, num_priors - 1)

    zi = jnp.zeros((16,), jnp.int32)
    ones = jnp.ones((16,), jnp.int32)

    def zero12(j, carry):
        h1c[pl.ds(j * 16, 16)] = zi
        h2c[pl.ds(j * 16, 16)] = zi
        return carry

    lax.fori_loop(0, 128, zero12, 0)

    def zero3(j, carry):
        h3c[pl.ds(j * 16, 16)] = zi
        return carry

    lax.fori_loop(0, 32, zero3, 0)

    def pass1(j, carry):
        for u in range(4):
            x = sv[pl.ds(j * 64 + u * 16, 16)]
            b = plsc.bitcast(x, jnp.int32)
            plsc.addupdate_scatter(h1c, [lax.shift_right_logical(b, 20)],
                                   ones)
        return carry

    lax.fori_loop(0, nchunk4, pass1, 0)

    def scan_level(hc, nchunks, cnt0):
        def body(t, carry):
            done, bsel, cnt, base = carry
            j = nchunks - 1 - t
            c = hc[pl.ds(j * 16, 16)]
            suf_c = suffix_i(c)
            mask = (cnt + suf_c) >= k
            nm = allsum_i(mask.astype(jnp.int32))
            found = jnp.logical_and(done == 0, nm > 0)
            istar = nm - 1
            isafe = jnp.minimum(jnp.maximum(istar, 0) + 1, 15)
            tot = pick_i(suf_c, zeros16)
            above_c = jnp.where(istar >= 15, 0, pick_i(suf_c, isafe))
            nb = jnp.where(found, base + istar, bsel)
            ncnt = jnp.where(found, cnt + above_c,
                             jnp.where(done == 0, cnt + tot, cnt))
            ndone = jnp.where(found, 1, done)
            return (ndone, nb, ncnt, base - 16)

        base0 = jnp.full((16,), (nchunks - 1) * 16, jnp.int32)
        out = lax.fori_loop(0, nchunks, body,
                            (zeros16, zeros16, cnt0, base0))
        return out[1], out[2]

    b1, cnt1 = scan_level(h1c, 128, zeros16)

    def pass2(j, carry):
        for u in range(4):
            x = sv[pl.ds(j * 64 + u * 16, 16)]
            b = plsc.bitcast(x, jnp.int32)
            msk = lax.shift_right_logical(b, 20) == b1
            i2 = jnp.bitwise_and(lax.shift_right_logical(b, 9), 0x7FF)
            plsc.addupdate_scatter(h2c, [i2], ones, mask=msk)
        return carry

    lax.fori_loop(0, nchunk4, pass2, 0)
    b2, cnt2 = scan_level(h2c, 128, cnt1)

    pref = jnp.bitwise_or(lax.shift_left(b1, 11), b2)

    def pass3(j, carry):
        for u in range(4):
            x = sv[pl.ds(j * 64 + u * 16, 16)]
            b = plsc.bitcast(x, jnp.int32)
            msk = lax.shift_right_logical(b, 9) == pref
            i3 = jnp.bitwise_and(b, 0x1FF)
            plsc.addupdate_scatter(h3c, [i3], ones, mask=msk)
        return carry

    lax.fori_loop(0, nchunk4, pass3, 0)
    b3, _ = scan_level(h3c, 32, cnt2)

    v_bits = jnp.bitwise_or(jnp.bitwise_or(
        lax.shift_left(b1, 20), lax.shift_left(b2, 9)), b3)

    rv[...] = v_bits
    pltpu.sync_copy(rv.at[pl.ds(0, 8)], out_hbm.at[pl.ds(row * 8, 8)])


def _fin_body(num_priors, s_ref, bits_ref, vb_ref, npos_ref, acc_ref,
              out_l_ref, out_c_ref):
    s = s_ref[...]
    bits = bits_ref[...]
    v = vb_ref[:, :1]
    num_pos = npos_ref[:, :1]
    num_neg = jnp.minimum(3 * num_pos.astype(jnp.int32), num_priors - 1)

    c_gt = jnp.sum((bits > v).astype(jnp.int32), axis=1, keepdims=True)
    sum_gt = jnp.sum(jnp.where(bits > v, s, 0.0), axis=1, keepdims=True)
    v_f = jnp.max(jnp.where(bits == v, s, 0.0), axis=1, keepdims=True)
    extra = jnp.where(num_neg > c_gt,
                      (num_neg - c_gt).astype(jnp.float32) * v_f, 0.0)
    neg_contrib = jnp.sum(sum_gt + extra)

    n_total = jnp.sum(num_pos)
    acc = acc_ref[...]
    lane = jax.lax.broadcasted_iota(jnp.int32, acc.shape, 1)
    loss_l_total = jnp.sum(jnp.where(lane == 0, acc, 0.0))
    pos_ce_total = jnp.sum(jnp.where(lane == 1, acc, 0.0))
    loss_l = loss_l_total / n_total
    loss_c = (pos_ce_total + neg_contrib) / n_total
    out_l_ref[...] = jnp.full(out_l_ref.shape, loss_l)
    out_c_ref[...] = jnp.full(out_c_ref.shape, loss_c)


def kernel(loc_t, loc_data, conf_t, conf_data):
    num, num_priors, nc = conf_data.shape
    ablocks = -(-num_priors // _AL)
    p_pad = ablocks * _AL
    pad = p_pad - num_priors

    cd = jnp.pad(jnp.transpose(conf_data, (0, 2, 1)),
                 ((0, 0), (0, 0), (0, pad))).astype(jnp.bfloat16)
    lt = jnp.pad(jnp.transpose(loc_t, (0, 2, 1)), ((0, 0), (0, 0), (0, pad)))
    ld = jnp.pad(jnp.transpose(loc_data, (0, 2, 1)), ((0, 0), (0, 0), (0, pad)))
    ct = jnp.pad(conf_t, ((0, 0), (0, pad))).reshape(num, 1, p_pad)

    s_out, npos_out, acc_out = pl.pallas_call(
        functools.partial(_dense_body, ablocks, num_priors),
        grid=(num, ablocks),
        in_specs=[
            pl.BlockSpec((1, nc, _AL), lambda b, a: (b, 0, a)),
            pl.BlockSpec((1, 1, _AL), lambda b, a: (b, 0, a)),
            pl.BlockSpec((1, 4, _AL), lambda b, a: (b, 0, a)),
            pl.BlockSpec((1, 4, _AL), lambda b, a: (b, 0, a)),
        ],
        out_specs=[
            pl.BlockSpec((1, 1, _AL), lambda b, a: (b, 0, a)),
            pl.BlockSpec((1, 1, 128), lambda b, a: (b, 0, 0)),
            pl.BlockSpec((1, 128), lambda b, a: (0, 0)),
        ],
        out_shape=[
            jax.ShapeDtypeStruct((num, 1, p_pad), jnp.float32),
            jax.ShapeDtypeStruct((num, 1, 128), jnp.float32),
            jax.ShapeDtypeStruct((1, 128), jnp.float32),
        ],
    )(cd, ct, lt, ld)

    s_flat = s_out.reshape(num * p_pad)
    npos = npos_out.reshape(num, 128)
    npos_i32 = npos[:, 0].astype(jnp.int32)

    mesh = plsc.VectorSubcoreMesh(core_axis_name="c", subcore_axis_name="s")
    vbits = pl.kernel(
        functools.partial(_sc_mine_body, p_pad, num_priors),
        mesh=mesh,
        compiler_params=pltpu.CompilerParams(needs_layout_passes=False),
        out_type=jax.ShapeDtypeStruct((num * 8,), jnp.int32),
        scratch_types=[
            pltpu.VMEM((p_pad,), jnp.float32),
            pltpu.VMEM((num,), jnp.int32),
            pltpu.VMEM((2048,), jnp.int32),
            pltpu.VMEM((2048,), jnp.int32),
            pltpu.VMEM((512,), jnp.int32),
            pltpu.VMEM((16,), jnp.int32),
            pltpu.VMEM((16,), jnp.int32),
        ],
    )(s_flat, npos_i32)

    s = s_out.reshape(num, p_pad)
    bits = jax.lax.bitcast_convert_type(s, jnp.int32)

    out_l, out_c = pl.pallas_call(
        functools.partial(_fin_body, num_priors),
        out_shape=[
            jax.ShapeDtypeStruct((1, 128), jnp.float32),
            jax.ShapeDtypeStruct((1, 128), jnp.float32),
        ],
    )(s, bits, vbits.reshape(num, 8), npos, acc_out)

    return (out_l[0, 0], out_c[0, 0])


# speedup vs baseline: 1.5065x; 1.1624x over previous
import functools

import jax
import jax.numpy as jnp
from jax import lax
from jax.experimental import pallas as pl
from jax.experimental.pallas import tpu as pltpu, tpu_sc as plsc

_AL = 4096
_NC = 81


def _dense_body(ablocks, num_priors, cd_ref, ct_ref, lt_ref, ld_ref,
                s_ref, npos_ref, acc_ref):
    b = pl.program_id(0)
    a = pl.program_id(1)

    @pl.when(jnp.logical_and(b == 0, a == 0))
    def _():
        acc_ref[...] = jnp.zeros_like(acc_ref)

    @pl.when(a == 0)
    def _():
        npos_ref[...] = jnp.zeros_like(npos_ref)

    x = cd_ref[0].astype(jnp.float32)
    ct = ct_ref[0]
    m = jnp.max(x, axis=0, keepdims=True)
    e = jnp.exp(x - m)
    lse = jnp.log(jnp.sum(e, axis=0, keepdims=True)) + m
    sub_iota = jax.lax.broadcasted_iota(jnp.int32, x.shape, 0)
    tgt = jnp.sum(jnp.where(sub_iota == ct, x, 0.0), axis=0, keepdims=True)
    ce = lse - tgt

    pos = ct > 0
    posf = pos.astype(jnp.float32)
    ai = jax.lax.broadcasted_iota(jnp.int32, ct.shape, 1) + a * _AL
    dead = jnp.logical_or(pos, ai >= num_priors)
    s_ref[0] = jnp.where(dead, 0.0, ce)

    diff = (ld_ref[0].astype(jnp.float32)
            - lt_ref[0].astype(jnp.float32))
    ad = jnp.abs(diff)
    sl1 = jnp.where(ad < 1.0, 0.5 * diff * diff, ad - 0.5)
    loss_l_part = jnp.sum(jnp.sum(sl1, axis=0, keepdims=True) * posf)
    pos_ce_part = jnp.sum(ce * posf)
    npos_part = jnp.sum(posf)

    npos_ref[0] += npos_part
    lane = jax.lax.broadcasted_iota(jnp.int32, (1, 128), 1)
    acc_ref[...] += (jnp.where(lane == 0, loss_l_part, 0.0)
                     + jnp.where(lane == 1, pos_ce_part, 0.0))


def _sc_mine_body(p_pad, num_priors,
                  s_hbm, npos_hbm, out_hbm,
                  sv, nv, h1c, h2c, h3c, rv, tmpi):
    row = lax.axis_index("s") * 2 + lax.axis_index("c")
    nchunk4 = p_pad // 64

    pltpu.sync_copy(s_hbm.at[pl.ds(row * p_pad, p_pad)], sv)
    pltpu.sync_copy(npos_hbm, nv)

    iota = lax.iota(jnp.int32, 16)
    zeros16 = jnp.zeros((16,), jnp.int32)

    def allsum_i(v):
        for d in (1, 2, 4, 8):
            tmpi[...] = v
            v = v + plsc.load_gather(tmpi, [jnp.bitwise_xor(iota, d)])
        return v

    def suffix_i(v):
        for d in (1, 2, 4, 8):
            tmpi[...] = v
            g = plsc.load_gather(tmpi, [jnp.minimum(iota + d, 15)])
            v = v + jnp.where(iota + d < 16, g, 0)
        return v

    def pick_i(v, idx):
        tmpi[...] = v
        return plsc.load_gather(tmpi, [idx])

    npos_w = plsc.load_gather(nv, [zeros16 + row])
    k = jnp.minimum(3 * npos_w, num_priors - 1)

    zi = jnp.zeros((16,), jnp.int32)
    ones = jnp.ones((16,), jnp.int32)

    def zero12(j, carry):
        h1c[pl.ds(j * 16, 16)] = zi
        h2c[pl.ds(j * 16, 16)] = zi
        return carry

    lax.fori_loop(0, 128, zero12, 0)

    def zero3(j, carry):
        h3c[pl.ds(j * 16, 16)] = zi
        return carry

    lax.fori_loop(0, 32, zero3, 0)

    def pass1(j, carry):
        for u in range(4):
            x = sv[pl.ds(j * 64 + u * 16, 16)]
            b = plsc.bitcast(x, jnp.int32)
            plsc.addupdate_scatter(h1c, [lax.shift_right_logical(b, 20)],
                                   ones)
        return carry

    lax.fori_loop(0, nchunk4, pass1, 0)

    def scan_level(hc, nchunks, cnt0):
        def body(t, carry):
            done, bsel, cnt, base = carry
            j = nchunks - 1 - t
            c = hc[pl.ds(j * 16, 16)]
            suf_c = suffix_i(c)
            mask = (cnt + suf_c) >= k
            nm = allsum_i(mask.astype(jnp.int32))
            found = jnp.logical_and(done == 0, nm > 0)
            istar = nm - 1
            isafe = jnp.minimum(jnp.maximum(istar, 0) + 1, 15)
            tot = pick_i(suf_c, zeros16)
            above_c = jnp.where(istar >= 15, 0, pick_i(suf_c, isafe))
            nb = jnp.where(found, base + istar, bsel)
            ncnt = jnp.where(found, cnt + above_c,
                             jnp.where(done == 0, cnt + tot, cnt))
            ndone = jnp.where(found, 1, done)
            return (ndone, nb, ncnt, base - 16)

        base0 = jnp.full((16,), (nchunks - 1) * 16, jnp.int32)
        out = lax.fori_loop(0, nchunks, body,
                            (zeros16, zeros16, cnt0, base0))
        return out[1], out[2]

    b1, cnt1 = scan_level(h1c, 128, zeros16)

    def pass2(j, carry):
        for u in range(4):
            x = sv[pl.ds(j * 64 + u * 16, 16)]
            b = plsc.bitcast(x, jnp.int32)
            msk = lax.shift_right_logical(b, 20) == b1
            i2 = jnp.bitwise_and(lax.shift_right_logical(b, 9), 0x7FF)
            plsc.addupdate_scatter(h2c, [i2], ones, mask=msk)
        return carry

    lax.fori_loop(0, nchunk4, pass2, 0)
    b2, cnt2 = scan_level(h2c, 128, cnt1)

    pref = jnp.bitwise_or(lax.shift_left(b1, 11), b2)

    def pass3(j, carry):
        for u in range(4):
            x = sv[pl.ds(j * 64 + u * 16, 16)]
            b = plsc.bitcast(x, jnp.int32)
            msk = lax.shift_right_logical(b, 9) == pref
            i3 = jnp.bitwise_and(b, 0x1FF)
            plsc.addupdate_scatter(h3c, [i3], ones, mask=msk)
        return carry

    lax.fori_loop(0, nchunk4, pass3, 0)
    b3, _ = scan_level(h3c, 32, cnt2)

    v_bits = jnp.bitwise_or(jnp.bitwise_or(
        lax.shift_left(b1, 20), lax.shift_left(b2, 9)), b3)

    rv[...] = v_bits
    pltpu.sync_copy(rv.at[pl.ds(0, 8)], out_hbm.at[pl.ds(row * 8, 8)])


def _fin_body(num_priors, s_ref, bits_ref, vb_ref, npos_ref, acc_ref,
              out_l_ref, out_c_ref):
    s = s_ref[...]
    bits = bits_ref[...]
    v = vb_ref[:, :1]
    num_pos = npos_ref[:, :1]
    num_neg = jnp.minimum(3 * num_pos.astype(jnp.int32), num_priors - 1)

    c_gt = jnp.sum((bits > v).astype(jnp.int32), axis=1, keepdims=True)
    sum_gt = jnp.sum(jnp.where(bits > v, s, 0.0), axis=1, keepdims=True)
    v_f = jnp.max(jnp.where(bits == v, s, 0.0), axis=1, keepdims=True)
    extra = jnp.where(num_neg > c_gt,
                      (num_neg - c_gt).astype(jnp.float32) * v_f, 0.0)
    neg_contrib = jnp.sum(sum_gt + extra)

    n_total = jnp.sum(num_pos)
    acc = acc_ref[...]
    lane = jax.lax.broadcasted_iota(jnp.int32, acc.shape, 1)
    loss_l_total = jnp.sum(jnp.where(lane == 0, acc, 0.0))
    pos_ce_total = jnp.sum(jnp.where(lane == 1, acc, 0.0))
    loss_l = loss_l_total / n_total
    loss_c = (pos_ce_total + neg_contrib) / n_total
    out_l_ref[...] = jnp.full(out_l_ref.shape, loss_l)
    out_c_ref[...] = jnp.full(out_c_ref.shape, loss_c)


def kernel(loc_t, loc_data, conf_t, conf_data):
    num, num_priors, nc = conf_data.shape
    ablocks = -(-num_priors // _AL)
    p_pad = ablocks * _AL
    pad = p_pad - num_priors

    cd = jnp.pad(jnp.transpose(conf_data, (0, 2, 1)),
                 ((0, 0), (0, 0), (0, pad))).astype(jnp.bfloat16)
    lt = jnp.pad(jnp.transpose(loc_t, (0, 2, 1)),
                 ((0, 0), (0, 0), (0, pad))).astype(jnp.bfloat16)
    ld = jnp.pad(jnp.transpose(loc_data, (0, 2, 1)),
                 ((0, 0), (0, 0), (0, pad))).astype(jnp.bfloat16)
    ct = jnp.pad(conf_t, ((0, 0), (0, pad))).reshape(num, 1, p_pad)

    s_out, npos_out, acc_out = pl.pallas_call(
        functools.partial(_dense_body, ablocks, num_priors),
        grid=(num, ablocks),
        in_specs=[
            pl.BlockSpec((1, nc, _AL), lambda b, a: (b, 0, a)),
            pl.BlockSpec((1, 1, _AL), lambda b, a: (b, 0, a)),
            pl.BlockSpec((1, 4, _AL), lambda b, a: (b, 0, a)),
            pl.BlockSpec((1, 4, _AL), lambda b, a: (b, 0, a)),
        ],
        out_specs=[
            pl.BlockSpec((1, 1, _AL), lambda b, a: (b, 0, a)),
            pl.BlockSpec((1, 1, 128), lambda b, a: (b, 0, 0)),
            pl.BlockSpec((1, 128), lambda b, a: (0, 0)),
        ],
        out_shape=[
            jax.ShapeDtypeStruct((num, 1, p_pad), jnp.float32),
            jax.ShapeDtypeStruct((num, 1, 128), jnp.float32),
            jax.ShapeDtypeStruct((1, 128), jnp.float32),
        ],
    )(cd, ct, lt, ld)

    s_flat = s_out.reshape(num * p_pad)
    npos = npos_out.reshape(num, 128)
    npos_i32 = npos[:, 0].astype(jnp.int32)

    mesh = plsc.VectorSubcoreMesh(core_axis_name="c", subcore_axis_name="s")
    vbits = pl.kernel(
        functools.partial(_sc_mine_body, p_pad, num_priors),
        mesh=mesh,
        compiler_params=pltpu.CompilerParams(needs_layout_passes=False),
        out_type=jax.ShapeDtypeStruct((num * 8,), jnp.int32),
        scratch_types=[
            pltpu.VMEM((p_pad,), jnp.float32),
            pltpu.VMEM((num,), jnp.int32),
            pltpu.VMEM((2048,), jnp.int32),
            pltpu.VMEM((2048,), jnp.int32),
            pltpu.VMEM((512,), jnp.int32),
            pltpu.VMEM((16,), jnp.int32),
            pltpu.VMEM((16,), jnp.int32),
        ],
    )(s_flat, npos_i32)

    s = s_out.reshape(num, p_pad)
    bits = jax.lax.bitcast_convert_type(s, jnp.int32)

    out_l, out_c = pl.pallas_call(
        functools.partial(_fin_body, num_priors),
        out_shape=[
            jax.ShapeDtypeStruct((1, 128), jnp.float32),
            jax.ShapeDtypeStruct((1, 128), jnp.float32),
        ],
    )(s, bits, vbits.reshape(num, 8), npos, acc_out)

    return (out_l[0, 0], out_c[0, 0])


# speedup vs baseline: 1.5869x; 1.0534x over previous
import functools

import jax
import jax.numpy as jnp
from jax import lax
from jax.experimental import pallas as pl
from jax.experimental.pallas import tpu as pltpu, tpu_sc as plsc

_AL = 10240
_NC = 81


def _dense_body(ablocks, num_priors, cd_ref, ct_ref, lt_ref, ld_ref,
                s_ref, npos_ref, acc_ref):
    b = pl.program_id(0)
    a = pl.program_id(1)

    @pl.when(jnp.logical_and(b == 0, a == 0))
    def _():
        acc_ref[...] = jnp.zeros_like(acc_ref)

    @pl.when(a == 0)
    def _():
        npos_ref[...] = jnp.zeros_like(npos_ref)

    x = cd_ref[0].astype(jnp.float32)
    ct = ct_ref[0]
    m = jnp.max(x, axis=0, keepdims=True)
    e = jnp.exp(x - m)
    lse = jnp.log(jnp.sum(e, axis=0, keepdims=True)) + m
    sub_iota = jax.lax.broadcasted_iota(jnp.int32, x.shape, 0)
    tgt = jnp.sum(jnp.where(sub_iota == ct, x, 0.0), axis=0, keepdims=True)
    ce = lse - tgt

    pos = ct > 0
    posf = pos.astype(jnp.float32)
    ai = jax.lax.broadcasted_iota(jnp.int32, ct.shape, 1) + a * _AL
    dead = jnp.logical_or(pos, ai >= num_priors)
    s_ref[0] = jnp.where(dead, 0.0, ce)

    diff = (ld_ref[0].astype(jnp.float32)
            - lt_ref[0].astype(jnp.float32))
    ad = jnp.abs(diff)
    sl1 = jnp.where(ad < 1.0, 0.5 * diff * diff, ad - 0.5)
    loss_l_part = jnp.sum(jnp.sum(sl1, axis=0, keepdims=True) * posf)
    pos_ce_part = jnp.sum(ce * posf)
    npos_part = jnp.sum(posf)

    npos_ref[0] += npos_part
    lane = jax.lax.broadcasted_iota(jnp.int32, (1, 128), 1)
    acc_ref[...] += (jnp.where(lane == 0, loss_l_part, 0.0)
                     + jnp.where(lane == 1, pos_ce_part, 0.0))


def _sc_mine_body(p_pad, num_priors,
                  s_hbm, npos_hbm, out_hbm,
                  sv, nv, h1c, h2c, h3c, rv, tmpi):
    row = lax.axis_index("s") * 2 + lax.axis_index("c")
    nchunk4 = p_pad // 64

    pltpu.sync_copy(s_hbm.at[pl.ds(row * p_pad, p_pad)], sv)
    pltpu.sync_copy(npos_hbm, nv)

    iota = lax.iota(jnp.int32, 16)
    zeros16 = jnp.zeros((16,), jnp.int32)

    def allsum_i(v):
        for d in (1, 2, 4, 8):
            tmpi[...] = v
            v = v + plsc.load_gather(tmpi, [jnp.bitwise_xor(iota, d)])
        return v

    def suffix_i(v):
        for d in (1, 2, 4, 8):
            tmpi[...] = v
            g = plsc.load_gather(tmpi, [jnp.minimum(iota + d, 15)])
            v = v + jnp.where(iota + d < 16, g, 0)
        return v

    def pick_i(v, idx):
        tmpi[...] = v
        return plsc.load_gather(tmpi, [idx])

    npos_w = plsc.load_gather(nv, [zeros16 + row])
    k = jnp.minimum(3 * npos_w, num_priors - 1)

    zi = jnp.zeros((16,), jnp.int32)
    ones = jnp.ones((16,), jnp.int32)

    def zero12(j, carry):
        h1c[pl.ds(j * 16, 16)] = zi
        h2c[pl.ds(j * 16, 16)] = zi
        return carry

    lax.fori_loop(0, 128, zero12, 0)

    def zero3(j, carry):
        h3c[pl.ds(j * 16, 16)] = zi
        return carry

    lax.fori_loop(0, 32, zero3, 0)

    def pass1(j, carry):
        for u in range(4):
            x = sv[pl.ds(j * 64 + u * 16, 16)]
            b = plsc.bitcast(x, jnp.int32)
            plsc.addupdate_scatter(h1c, [lax.shift_right_logical(b, 20)],
                                   ones)
        return carry

    lax.fori_loop(0, nchunk4, pass1, 0)

    def scan_level(hc, nchunks, cnt0):
        def body(t, carry):
            done, bsel, cnt, base = carry
            j = nchunks - 1 - t
            c = hc[pl.ds(j * 16, 16)]
            suf_c = suffix_i(c)
            mask = (cnt + suf_c) >= k
            nm = allsum_i(mask.astype(jnp.int32))
            found = jnp.logical_and(done == 0, nm > 0)
            istar = nm - 1
            isafe = jnp.minimum(jnp.maximum(istar, 0) + 1, 15)
            tot = pick_i(suf_c, zeros16)
            above_c = jnp.where(istar >= 15, 0, pick_i(suf_c, isafe))
            nb = jnp.where(found, base + istar, bsel)
            ncnt = jnp.where(found, cnt + above_c,
                             jnp.where(done == 0, cnt + tot, cnt))
            ndone = jnp.where(found, 1, done)
            return (ndone, nb, ncnt, base - 16)

        base0 = jnp.full((16,), (nchunks - 1) * 16, jnp.int32)
        out = lax.fori_loop(0, nchunks, body,
                            (zeros16, zeros16, cnt0, base0))
        return out[1], out[2]

    b1, cnt1 = scan_level(h1c, 128, zeros16)

    def pass2(j, carry):
        for u in range(4):
            x = sv[pl.ds(j * 64 + u * 16, 16)]
            b = plsc.bitcast(x, jnp.int32)
            msk = lax.shift_right_logical(b, 20) == b1
            i2 = jnp.bitwise_and(lax.shift_right_logical(b, 9), 0x7FF)
            plsc.addupdate_scatter(h2c, [i2], ones, mask=msk)
        return carry

    lax.fori_loop(0, nchunk4, pass2, 0)
    b2, cnt2 = scan_level(h2c, 128, cnt1)

    pref = jnp.bitwise_or(lax.shift_left(b1, 11), b2)

    def pass3(j, carry):
        for u in range(4):
            x = sv[pl.ds(j * 64 + u * 16, 16)]
            b = plsc.bitcast(x, jnp.int32)
            msk = lax.shift_right_logical(b, 9) == pref
            i3 = jnp.bitwise_and(b, 0x1FF)
            plsc.addupdate_scatter(h3c, [i3], ones, mask=msk)
        return carry

    lax.fori_loop(0, nchunk4, pass3, 0)
    b3, _ = scan_level(h3c, 32, cnt2)

    v_bits = jnp.bitwise_or(jnp.bitwise_or(
        lax.shift_left(b1, 20), lax.shift_left(b2, 9)), b3)

    rv[...] = v_bits
    pltpu.sync_copy(rv.at[pl.ds(0, 8)], out_hbm.at[pl.ds(row * 8, 8)])


def _fin_body(num_priors, s_ref, bits_ref, vb_ref, npos_ref, acc_ref,
              out_l_ref, out_c_ref):
    s = s_ref[...]
    bits = bits_ref[...]
    v = vb_ref[:, :1]
    num_pos = npos_ref[:, :1]
    num_neg = jnp.minimum(3 * num_pos.astype(jnp.int32), num_priors - 1)

    c_gt = jnp.sum((bits > v).astype(jnp.int32), axis=1, keepdims=True)
    sum_gt = jnp.sum(jnp.where(bits > v, s, 0.0), axis=1, keepdims=True)
    v_f = jnp.max(jnp.where(bits == v, s, 0.0), axis=1, keepdims=True)
    extra = jnp.where(num_neg > c_gt,
                      (num_neg - c_gt).astype(jnp.float32) * v_f, 0.0)
    neg_contrib = jnp.sum(sum_gt + extra)

    n_total = jnp.sum(num_pos)
    acc = acc_ref[...]
    lane = jax.lax.broadcasted_iota(jnp.int32, acc.shape, 1)
    loss_l_total = jnp.sum(jnp.where(lane == 0, acc, 0.0))
    pos_ce_total = jnp.sum(jnp.where(lane == 1, acc, 0.0))
    loss_l = loss_l_total / n_total
    loss_c = (pos_ce_total + neg_contrib) / n_total
    out_l_ref[...] = jnp.full(out_l_ref.shape, loss_l)
    out_c_ref[...] = jnp.full(out_c_ref.shape, loss_c)


def kernel(loc_t, loc_data, conf_t, conf_data):
    num, num_priors, nc = conf_data.shape
    ablocks = -(-num_priors // _AL)
    p_pad = ablocks * _AL
    pad = p_pad - num_priors

    cd = jnp.pad(jnp.transpose(conf_data, (0, 2, 1)),
                 ((0, 0), (0, 0), (0, pad))).astype(jnp.bfloat16)
    lt = jnp.pad(jnp.transpose(loc_t, (0, 2, 1)),
                 ((0, 0), (0, 0), (0, pad))).astype(jnp.bfloat16)
    ld = jnp.pad(jnp.transpose(loc_data, (0, 2, 1)),
                 ((0, 0), (0, 0), (0, pad))).astype(jnp.bfloat16)
    ct = jnp.pad(conf_t, ((0, 0), (0, pad))).reshape(num, 1, p_pad)

    s_out, npos_out, acc_out = pl.pallas_call(
        functools.partial(_dense_body, ablocks, num_priors),
        grid=(num, ablocks),
        in_specs=[
            pl.BlockSpec((1, nc, _AL), lambda b, a: (b, 0, a)),
            pl.BlockSpec((1, 1, _AL), lambda b, a: (b, 0, a)),
            pl.BlockSpec((1, 4, _AL), lambda b, a: (b, 0, a)),
            pl.BlockSpec((1, 4, _AL), lambda b, a: (b, 0, a)),
        ],
        out_specs=[
            pl.BlockSpec((1, 1, _AL), lambda b, a: (b, 0, a)),
            pl.BlockSpec((1, 1, 128), lambda b, a: (b, 0, 0)),
            pl.BlockSpec((1, 128), lambda b, a: (0, 0)),
        ],
        out_shape=[
            jax.ShapeDtypeStruct((num, 1, p_pad), jnp.float32),
            jax.ShapeDtypeStruct((num, 1, 128), jnp.float32),
            jax.ShapeDtypeStruct((1, 128), jnp.float32),
        ],
    )(cd, ct, lt, ld)

    s_flat = s_out.reshape(num * p_pad)
    npos = npos_out.reshape(num, 128)
    npos_i32 = npos[:, 0].astype(jnp.int32)

    mesh = plsc.VectorSubcoreMesh(core_axis_name="c", subcore_axis_name="s")
    vbits = pl.kernel(
        functools.partial(_sc_mine_body, p_pad, num_priors),
        mesh=mesh,
        compiler_params=pltpu.CompilerParams(needs_layout_passes=False),
        out_type=jax.ShapeDtypeStruct((num * 8,), jnp.int32),
        scratch_types=[
            pltpu.VMEM((p_pad,), jnp.float32),
            pltpu.VMEM((num,), jnp.int32),
            pltpu.VMEM((2048,), jnp.int32),
            pltpu.VMEM((2048,), jnp.int32),
            pltpu.VMEM((512,), jnp.int32),
            pltpu.VMEM((16,), jnp.int32),
            pltpu.VMEM((16,), jnp.int32),
        ],
    )(s_flat, npos_i32)

    s = s_out.reshape(num, p_pad)
    bits = jax.lax.bitcast_convert_type(s, jnp.int32)

    out_l, out_c = pl.pallas_call(
        functools.partial(_fin_body, num_priors),
        out_shape=[
            jax.ShapeDtypeStruct((1, 128), jnp.float32),
            jax.ShapeDtypeStruct((1, 128), jnp.float32),
        ],
    )(s, bits, vbits.reshape(num, 8), npos, acc_out)

    return (out_l[0, 0], out_c[0, 0])


# speedup vs baseline: 1.5913x; 1.0027x over previous
import functools

import jax
import jax.numpy as jnp
from jax import lax
from jax.experimental import pallas as pl
from jax.experimental.pallas import tpu as pltpu, tpu_sc as plsc

_AL = 20480
_NC = 81


def _dense_body(ablocks, num_priors, cd_ref, ct_ref, lt_ref, ld_ref,
                s_ref, npos_ref, acc_ref):
    b = pl.program_id(0)
    a = pl.program_id(1)

    @pl.when(jnp.logical_and(b == 0, a == 0))
    def _():
        acc_ref[...] = jnp.zeros_like(acc_ref)

    @pl.when(a == 0)
    def _():
        npos_ref[...] = jnp.zeros_like(npos_ref)

    x = cd_ref[0].astype(jnp.float32)
    ct = ct_ref[0]
    m = jnp.max(x, axis=0, keepdims=True)
    e = jnp.exp(x - m)
    lse = jnp.log(jnp.sum(e, axis=0, keepdims=True)) + m
    sub_iota = jax.lax.broadcasted_iota(jnp.int32, x.shape, 0)
    tgt = jnp.sum(jnp.where(sub_iota == ct, x, 0.0), axis=0, keepdims=True)
    ce = lse - tgt

    pos = ct > 0
    posf = pos.astype(jnp.float32)
    ai = jax.lax.broadcasted_iota(jnp.int32, ct.shape, 1) + a * _AL
    dead = jnp.logical_or(pos, ai >= num_priors)
    s_ref[0] = jnp.where(dead, 0.0, ce)

    diff = (ld_ref[0].astype(jnp.float32)
            - lt_ref[0].astype(jnp.float32))
    ad = jnp.abs(diff)
    sl1 = jnp.where(ad < 1.0, 0.5 * diff * diff, ad - 0.5)
    loss_l_part = jnp.sum(jnp.sum(sl1, axis=0, keepdims=True) * posf)
    pos_ce_part = jnp.sum(ce * posf)
    npos_part = jnp.sum(posf)

    npos_ref[0] += npos_part
    lane = jax.lax.broadcasted_iota(jnp.int32, (1, 128), 1)
    acc_ref[...] += (jnp.where(lane == 0, loss_l_part, 0.0)
                     + jnp.where(lane == 1, pos_ce_part, 0.0))


def _sc_mine_body(p_pad, num_priors,
                  s_hbm, npos_hbm, out_hbm,
                  sv, nv, h1c, h2c, h3c, rv, tmpi):
    row = lax.axis_index("s") * 2 + lax.axis_index("c")
    nchunk4 = p_pad // 64

    pltpu.sync_copy(s_hbm.at[pl.ds(row * p_pad, p_pad)], sv)
    pltpu.sync_copy(npos_hbm, nv)

    iota = lax.iota(jnp.int32, 16)
    zeros16 = jnp.zeros((16,), jnp.int32)

    def allsum_i(v):
        for d in (1, 2, 4, 8):
            tmpi[...] = v
            v = v + plsc.load_gather(tmpi, [jnp.bitwise_xor(iota, d)])
        return v

    def suffix_i(v):
        for d in (1, 2, 4, 8):
            tmpi[...] = v
            g = plsc.load_gather(tmpi, [jnp.minimum(iota + d, 15)])
            v = v + jnp.where(iota + d < 16, g, 0)
        return v

    def pick_i(v, idx):
        tmpi[...] = v
        return plsc.load_gather(tmpi, [idx])

    npos_w = plsc.load_gather(nv, [zeros16 + row])
    k = jnp.minimum(3 * npos_w, num_priors - 1)

    zi = jnp.zeros((16,), jnp.int32)
    ones = jnp.ones((16,), jnp.int32)

    def zero12(j, carry):
        h1c[pl.ds(j * 16, 16)] = zi
        h2c[pl.ds(j * 16, 16)] = zi
        return carry

    lax.fori_loop(0, 128, zero12, 0)

    def zero3(j, carry):
        h3c[pl.ds(j * 16, 16)] = zi
        return carry

    lax.fori_loop(0, 32, zero3, 0)

    def pass1(j, carry):
        for u in range(4):
            x = sv[pl.ds(j * 64 + u * 16, 16)]
            b = plsc.bitcast(x, jnp.int32)
            plsc.addupdate_scatter(h1c, [lax.shift_right_logical(b, 20)],
                                   ones)
        return carry

    lax.fori_loop(0, nchunk4, pass1, 0)

    def scan_level(hc, nchunks, cnt0):
        def body(t, carry):
            done, bsel, cnt, base = carry
            j = nchunks - 1 - t
            c = hc[pl.ds(j * 16, 16)]
            suf_c = suffix_i(c)
            mask = (cnt + suf_c) >= k
            nm = allsum_i(mask.astype(jnp.int32))
            found = jnp.logical_and(done == 0, nm > 0)
            istar = nm - 1
            isafe = jnp.minimum(jnp.maximum(istar, 0) + 1, 15)
            tot = pick_i(suf_c, zeros16)
            above_c = jnp.where(istar >= 15, 0, pick_i(suf_c, isafe))
            nb = jnp.where(found, base + istar, bsel)
            ncnt = jnp.where(found, cnt + above_c,
                             jnp.where(done == 0, cnt + tot, cnt))
            ndone = jnp.where(found, 1, done)
            return (ndone, nb, ncnt, base - 16)

        base0 = jnp.full((16,), (nchunks - 1) * 16, jnp.int32)
        out = lax.fori_loop(0, nchunks, body,
                            (zeros16, zeros16, cnt0, base0))
        return out[1], out[2]

    b1, cnt1 = scan_level(h1c, 128, zeros16)

    def pass2(j, carry):
        for u in range(4):
            x = sv[pl.ds(j * 64 + u * 16, 16)]
            b = plsc.bitcast(x, jnp.int32)
            msk = lax.shift_right_logical(b, 20) == b1
            i2 = jnp.bitwise_and(lax.shift_right_logical(b, 9), 0x7FF)
            plsc.addupdate_scatter(h2c, [i2], ones, mask=msk)
        return carry

    lax.fori_loop(0, nchunk4, pass2, 0)
    b2, cnt2 = scan_level(h2c, 128, cnt1)

    pref = jnp.bitwise_or(lax.shift_left(b1, 11), b2)

    def pass3(j, carry):
        for u in range(4):
            x = sv[pl.ds(j * 64 + u * 16, 16)]
            b = plsc.bitcast(x, jnp.int32)
            msk = lax.shift_right_logical(b, 9) == pref
            i3 = jnp.bitwise_and(b, 0x1FF)
            plsc.addupdate_scatter(h3c, [i3], ones, mask=msk)
        return carry

    lax.fori_loop(0, nchunk4, pass3, 0)
    b3, _ = scan_level(h3c, 32, cnt2)

    v_bits = jnp.bitwise_or(jnp.bitwise_or(
        lax.shift_left(b1, 20), lax.shift_left(b2, 9)), b3)

    rv[...] = v_bits
    pltpu.sync_copy(rv.at[pl.ds(0, 8)], out_hbm.at[pl.ds(row * 8, 8)])


def _fin_body(num_priors, s_ref, bits_ref, vb_ref, npos_ref, acc_ref,
              out_l_ref, out_c_ref):
    s = s_ref[...]
    bits = bits_ref[...]
    v = vb_ref[:, :1]
    num_pos = npos_ref[:, :1]
    num_neg = jnp.minimum(3 * num_pos.astype(jnp.int32), num_priors - 1)

    c_gt = jnp.sum((bits > v).astype(jnp.int32), axis=1, keepdims=True)
    sum_gt = jnp.sum(jnp.where(bits > v, s, 0.0), axis=1, keepdims=True)
    v_f = jnp.max(jnp.where(bits == v, s, 0.0), axis=1, keepdims=True)
    extra = jnp.where(num_neg > c_gt,
                      (num_neg - c_gt).astype(jnp.float32) * v_f, 0.0)
    neg_contrib = jnp.sum(sum_gt + extra)

    n_total = jnp.sum(num_pos)
    acc = acc_ref[...]
    lane = jax.lax.broadcasted_iota(jnp.int32, acc.shape, 1)
    loss_l_total = jnp.sum(jnp.where(lane == 0, acc, 0.0))
    pos_ce_total = jnp.sum(jnp.where(lane == 1, acc, 0.0))
    loss_l = loss_l_total / n_total
    loss_c = (pos_ce_total + neg_contrib) / n_total
    out_l_ref[...] = jnp.full(out_l_ref.shape, loss_l)
    out_c_ref[...] = jnp.full(out_c_ref.shape, loss_c)


def kernel(loc_t, loc_data, conf_t, conf_data):
    num, num_priors, nc = conf_data.shape
    ablocks = -(-num_priors // _AL)
    p_pad = ablocks * _AL
    pad = p_pad - num_priors

    cd = jnp.pad(jnp.transpose(conf_data, (0, 2, 1)),
                 ((0, 0), (0, 0), (0, pad))).astype(jnp.bfloat16)
    lt = jnp.pad(jnp.transpose(loc_t, (0, 2, 1)),
                 ((0, 0), (0, 0), (0, pad))).astype(jnp.bfloat16)
    ld = jnp.pad(jnp.transpose(loc_data, (0, 2, 1)),
                 ((0, 0), (0, 0), (0, pad))).astype(jnp.bfloat16)
    ct = jnp.pad(conf_t, ((0, 0), (0, pad))).reshape(num, 1, p_pad)

    s_out, npos_out, acc_out = pl.pallas_call(
        functools.partial(_dense_body, ablocks, num_priors),
        grid=(num, ablocks),
        in_specs=[
            pl.BlockSpec((1, nc, _AL), lambda b, a: (b, 0, a)),
            pl.BlockSpec((1, 1, _AL), lambda b, a: (b, 0, a)),
            pl.BlockSpec((1, 4, _AL), lambda b, a: (b, 0, a)),
            pl.BlockSpec((1, 4, _AL), lambda b, a: (b, 0, a)),
        ],
        out_specs=[
            pl.BlockSpec((1, 1, _AL), lambda b, a: (b, 0, a)),
            pl.BlockSpec((1, 1, 128), lambda b, a: (b, 0, 0)),
            pl.BlockSpec((1, 128), lambda b, a: (0, 0)),
        ],
        out_shape=[
            jax.ShapeDtypeStruct((num, 1, p_pad), jnp.float32),
            jax.ShapeDtypeStruct((num, 1, 128), jnp.float32),
            jax.ShapeDtypeStruct((1, 128), jnp.float32),
        ],
    )(cd, ct, lt, ld)

    s_flat = s_out.reshape(num * p_pad)
    npos = npos_out.reshape(num, 128)
    npos_i32 = npos[:, 0].astype(jnp.int32)

    mesh = plsc.VectorSubcoreMesh(core_axis_name="c", subcore_axis_name="s")
    vbits = pl.kernel(
        functools.partial(_sc_mine_body, p_pad, num_priors),
        mesh=mesh,
        compiler_params=pltpu.CompilerParams(needs_layout_passes=False),
        out_type=jax.ShapeDtypeStruct((num * 8,), jnp.int32),
        scratch_types=[
            pltpu.VMEM((p_pad,), jnp.float32),
            pltpu.VMEM((num,), jnp.int32),
            pltpu.VMEM((2048,), jnp.int32),
            pltpu.VMEM((2048,), jnp.int32),
            pltpu.VMEM((512,), jnp.int32),
            pltpu.VMEM((16,), jnp.int32),
            pltpu.VMEM((16,), jnp.int32),
        ],
    )(s_flat, npos_i32)

    s = s_out.reshape(num, p_pad)
    bits = jax.lax.bitcast_convert_type(s, jnp.int32)

    out_l, out_c = pl.pallas_call(
        functools.partial(_fin_body, num_priors),
        out_shape=[
            jax.ShapeDtypeStruct((1, 128), jnp.float32),
            jax.ShapeDtypeStruct((1, 128), jnp.float32),
        ],
    )(s, bits, vbits.reshape(num, 8), npos, acc_out)

    return (out_l[0, 0], out_c[0, 0])
